# baseline (speedup 1.0000x reference)
import functools

import jax
import jax.numpy as jnp
from jax.experimental import pallas as pl
from jax.experimental.pallas import tpu as pltpu


def _lse_lane_tiles(la, n_nodes, bb):
    mx = la[:, 0:bb]
    for i in range(1, n_nodes):
        mx = jnp.maximum(mx, la[:, i * bb:(i + 1) * bb])
    ex = jnp.exp(la - pltpu.repeat(mx, n_nodes, axis=1))
    sm = ex[:, 0:bb]
    for i in range(1, n_nodes):
        sm = sm + ex[:, i * bb:(i + 1) * bb]
    return pltpu.repeat(mx + jnp.log(sm).astype(la.dtype), n_nodes, axis=1)


def _lse_sublanes(la):
    mx = jnp.max(la, axis=0, keepdims=True)
    ex = jnp.exp(la - mx)
    sm = jnp.sum(ex, axis=0, keepdims=True, dtype=la.dtype)
    return mx + jnp.log(sm).astype(la.dtype)


def _nanl_kernel(x_hbm, sz_ref, w1_ref, b1_ref, w2_ref, b2_ref, out_ref,
                 xbuf, sems, *, n_nodes, n_feat, bb, inv_temp, sinkhorn_iters):
    N = n_nodes
    F = n_feat
    L = N * bb
    g = pl.program_id(0)
    n_blocks = pl.num_programs(0)
    slot = jax.lax.rem(g, 2)

    def issue(blk, buf):
        for s in range(2):
            for n in range(N):
                k = s * N + n
                pltpu.make_async_copy(
                    x_hbm.at[blk, :, s, n, :], xbuf.at[buf, k],
                    sems.at[buf, k]).start()

    @pl.when(g == 0)
    def _():
        issue(0, 0)

    @pl.when(g + 1 < n_blocks)
    def _():
        issue(g + 1, jax.lax.rem(g + 1, 2))

    for k in range(2 * N):
        pltpu.make_async_copy(xbuf.at[slot, k], xbuf.at[slot, k],
                              sems.at[slot, k]).wait()

    parts = []
    for k in range(2 * N):
        parts.append(xbuf[slot, k].T)
    X = jnp.concatenate(parts, axis=1)

    mparts = []
    for s in range(2):
        szs = sz_ref[0, s:s + 1, :]
        for n in range(N):
            mparts.append(jnp.where(float(n) < szs, 1.0, 0.0))
    mrow = jnp.concatenate(mparts, axis=1)

    h = jnp.maximum(
        jnp.dot(w1_ref[...], X, preferred_element_type=jnp.float32) + b1_ref[...], 0.0)
    e = jnp.dot(w2_ref[...], h, preferred_element_type=jnp.float32) + b2_ref[...]
    e = e * mrow

    ce = e[:, L:]
    rows = []
    for n in range(N):
        qr = pltpu.repeat(e[:, n * bb:(n + 1) * bb], N, axis=1)
        rows.append(jnp.sum(qr * ce, axis=0, keepdims=True))
    la = jnp.concatenate(rows, axis=0) * inv_temp

    la = la - _lse_lane_tiles(la, N, bb)
    la = la - _lse_sublanes(la)
    la = la.astype(jnp.bfloat16)
    for _ in range(1, sinkhorn_iters):
        la = la - _lse_lane_tiles(la, N, bb)
        la = la - _lse_sublanes(la)
    plan = jnp.exp(la)

    Xb = X.astype(jnp.bfloat16)
    pb = plan
    sc = jnp.zeros((1, bb), jnp.float32)
    for n in range(N):
        pn = pb[n:n + 1, :]
        recon = pn[:, 0:bb] * Xb[:, L:L + bb]
        for m in range(1, N):
            recon = recon + pn[:, m * bb:(m + 1) * bb] * Xb[:, L + m * bb:L + (m + 1) * bb]
        diff = jnp.maximum(Xb[:, n * bb:(n + 1) * bb] - recon, jnp.bfloat16(0.0))
        part = jnp.sum(diff, axis=0, keepdims=True, dtype=jnp.bfloat16)
        sc = sc + part.astype(jnp.float32)
    out_ref[...] = (-sc).reshape(1, 1, bb)


def _pick_batch_block(batch):
    for d in (128, 64, 32, 16, 8, 4, 2):
        if d <= batch and batch % d == 0 and batch // d >= 2:
            return d
    return batch


def _run_blocks(x, sz, w1t, b1c, w2t, b2c, *, n_nodes, n_feat, bb):
    G = x.shape[0]
    T = w1t.shape[0]
    F = n_feat
    N = n_nodes
    kern = functools.partial(_nanl_kernel, n_nodes=N, n_feat=F, bb=bb,
                             inv_temp=10.0, sinkhorn_iters=10)
    return pl.pallas_call(
        kern,
        grid=(G,),
        out_shape=jax.ShapeDtypeStruct((G, 1, bb), jnp.float32),
        in_specs=[
            pl.BlockSpec(memory_space=pl.ANY),
            pl.BlockSpec((1, 2, bb), lambda g: (g, 0, 0)),
            pl.BlockSpec((T, F), lambda g: (0, 0)),
            pl.BlockSpec((T, 1), lambda g: (0, 0)),
            pl.BlockSpec((T, T), lambda g: (0, 0)),
            pl.BlockSpec((T, 1), lambda g: (0, 0)),
        ],
        out_specs=pl.BlockSpec((1, 1, bb), lambda g: (g, 0, 0)),
        scratch_shapes=[
            pltpu.VMEM((2, 2 * N, bb, F), jnp.float32),
            pltpu.SemaphoreType.DMA((2, 2 * N)),
        ],
        compiler_params=pltpu.CompilerParams(
            dimension_semantics=("arbitrary",),
            vmem_limit_bytes=32 * 1024 * 1024),
    )(x, sz, w1t, b1c, w2t, b2c)


def kernel(stacked_qc, graph_sizes, w1, b1, w2, b2):
    B, two, N, F = stacked_qc.shape
    assert two == 2
    bb = _pick_batch_block(B)
    G = B // bb

    x = stacked_qc.astype(jnp.float32).reshape(G, bb, 2, N, F)
    sz = (graph_sizes.astype(jnp.float32)
          .reshape(G, bb, 2)
          .transpose(0, 2, 1))
    wargs = (w1.T.astype(jnp.float32), b1.reshape(-1, 1).astype(jnp.float32),
             w2.T.astype(jnp.float32), b2.reshape(-1, 1).astype(jnp.float32))

    run = functools.partial(_run_blocks, n_nodes=N, n_feat=F, bb=bb)
    out = run(x, sz, *wargs)
    return out.reshape(B)


# speedup vs baseline: 1.0379x; 1.0379x over previous
import functools

import jax
import jax.numpy as jnp
from jax.experimental import pallas as pl
from jax.experimental.pallas import tpu as pltpu


def _lse_lane_tiles(la, n_nodes, bb):
    mx = la[:, 0:bb]
    for i in range(1, n_nodes):
        mx = jnp.maximum(mx, la[:, i * bb:(i + 1) * bb])
    ex = jnp.exp(la - pltpu.repeat(mx, n_nodes, axis=1))
    sm = ex[:, 0:bb]
    for i in range(1, n_nodes):
        sm = sm + ex[:, i * bb:(i + 1) * bb]
    return pltpu.repeat(mx + jnp.log(sm), n_nodes, axis=1)


def _lse_sublanes(la):
    mx = jnp.max(la, axis=0, keepdims=True)
    ex = jnp.exp(la - mx)
    sm = jnp.sum(ex, axis=0, keepdims=True)
    return mx + jnp.log(sm)


def _nanl_kernel(x_hbm, sz_ref, w1_ref, b1_ref, w2_ref, b2_ref, out_ref,
                 xbuf, sems, *, n_nodes, n_feat, bb, inv_temp, sinkhorn_iters):
    N = n_nodes
    F = n_feat
    L = N * bb
    g = pl.program_id(0)
    n_blocks = pl.num_programs(0)
    slot = jax.lax.rem(g, 2)

    def issue(blk, buf):
        for s in range(2):
            for n in range(N):
                k = s * N + n
                pltpu.make_async_copy(
                    x_hbm.at[blk, :, s, n, :], xbuf.at[buf, k],
                    sems.at[buf]).start()

    @pl.when(g == 0)
    def _():
        issue(0, 0)

    @pl.when(g + 1 < n_blocks)
    def _():
        issue(g + 1, jax.lax.rem(g + 1, 2))

    for k in range(2 * N):
        pltpu.make_async_copy(xbuf.at[slot, k], xbuf.at[slot, k],
                              sems.at[slot]).wait()

    parts = []
    for k in range(2 * N):
        parts.append(xbuf[slot, k].T)
    X = jnp.concatenate(parts, axis=1)

    mparts = []
    for s in range(2):
        szs = sz_ref[0, s:s + 1, :]
        for n in range(N):
            mparts.append(jnp.where(float(n) < szs, 1.0, 0.0))
    mrow = jnp.concatenate(mparts, axis=1)

    h = jnp.maximum(
        jnp.dot(w1_ref[...], X, preferred_element_type=jnp.float32) + b1_ref[...], 0.0)
    e = jnp.dot(w2_ref[...], h, preferred_element_type=jnp.float32) + b2_ref[...]
    e = e * mrow

    ce = e[:, L:]
    rows = []
    for n in range(N):
        qr = pltpu.repeat(e[:, n * bb:(n + 1) * bb], N, axis=1)
        rows.append(jnp.sum(qr * ce, axis=0, keepdims=True))
    la = jnp.concatenate(rows, axis=0) * inv_temp

    for _ in range(sinkhorn_iters):
        la = la - _lse_lane_tiles(la, N, bb)
        la = la - _lse_sublanes(la)
    plan = jnp.exp(la)

    Xb = X.astype(jnp.bfloat16)
    pb = plan.astype(jnp.bfloat16)
    sc = jnp.zeros((1, bb), jnp.float32)
    for n in range(N):
        pn = pb[n:n + 1, :]
        recon = pn[:, 0:bb] * Xb[:, L:L + bb]
        for m in range(1, N):
            recon = recon + pn[:, m * bb:(m + 1) * bb] * Xb[:, L + m * bb:L + (m + 1) * bb]
        diff = jnp.maximum(Xb[:, n * bb:(n + 1) * bb] - recon, jnp.bfloat16(0.0))
        part = jnp.sum(diff, axis=0, keepdims=True, dtype=jnp.bfloat16)
        sc = sc + part.astype(jnp.float32)
    out_ref[...] = (-sc).reshape(1, 1, bb)


def _pick_batch_block(batch):
    for d in (128, 64, 32, 16, 8, 4, 2):
        if d <= batch and batch % d == 0 and batch // d >= 2:
            return d
    return batch


def _run_blocks(x, sz, w1t, b1c, w2t, b2c, *, n_nodes, n_feat, bb):
    G = x.shape[0]
    T = w1t.shape[0]
    F = n_feat
    N = n_nodes
    kern = functools.partial(_nanl_kernel, n_nodes=N, n_feat=F, bb=bb,
                             inv_temp=10.0, sinkhorn_iters=10)
    return pl.pallas_call(
        kern,
        grid=(G,),
        out_shape=jax.ShapeDtypeStruct((G, 1, bb), jnp.float32),
        in_specs=[
            pl.BlockSpec(memory_space=pl.ANY),
            pl.BlockSpec((1, 2, bb), lambda g: (g, 0, 0)),
            pl.BlockSpec((T, F), lambda g: (0, 0)),
            pl.BlockSpec((T, 1), lambda g: (0, 0)),
            pl.BlockSpec((T, T), lambda g: (0, 0)),
            pl.BlockSpec((T, 1), lambda g: (0, 0)),
        ],
        out_specs=pl.BlockSpec((1, 1, bb), lambda g: (g, 0, 0)),
        scratch_shapes=[
            pltpu.VMEM((2, 2 * N, bb, F), jnp.float32),
            pltpu.SemaphoreType.DMA((2,)),
        ],
        compiler_params=pltpu.CompilerParams(
            dimension_semantics=("arbitrary",),
            vmem_limit_bytes=32 * 1024 * 1024),
    )(x, sz, w1t, b1c, w2t, b2c)


def kernel(stacked_qc, graph_sizes, w1, b1, w2, b2):
    B, two, N, F = stacked_qc.shape
    assert two == 2
    bb = _pick_batch_block(B)
    G = B // bb

    x = stacked_qc.astype(jnp.float32).reshape(G, bb, 2, N, F)
    sz = (graph_sizes.astype(jnp.float32)
          .reshape(G, bb, 2)
          .transpose(0, 2, 1))
    wargs = (w1.T.astype(jnp.float32), b1.reshape(-1, 1).astype(jnp.float32),
             w2.T.astype(jnp.float32), b2.reshape(-1, 1).astype(jnp.float32))

    run = functools.partial(_run_blocks, n_nodes=N, n_feat=F, bb=bb)
    out = run(x, sz, *wargs)
    return out.reshape(B)


# speedup vs baseline: 1.0386x; 1.0007x over previous
import functools

import jax
import jax.numpy as jnp
from jax.experimental import pallas as pl
from jax.experimental.pallas import tpu as pltpu


def _lse_lane_tiles(la, n_nodes, bb):
    mx = la[:, 0:bb]
    for i in range(1, n_nodes):
        mx = jnp.maximum(mx, la[:, i * bb:(i + 1) * bb])
    ex = jnp.exp(la - pltpu.repeat(mx, n_nodes, axis=1))
    sm = ex[:, 0:bb]
    for i in range(1, n_nodes):
        sm = sm + ex[:, i * bb:(i + 1) * bb]
    return pltpu.repeat(mx + jnp.log(sm), n_nodes, axis=1)


def _lse_sublanes(la):
    mx = jnp.max(la, axis=0, keepdims=True)
    ex = jnp.exp(la - mx)
    sm = jnp.sum(ex, axis=0, keepdims=True)
    return mx + jnp.log(sm)


def _nanl_kernel(x_hbm, sz_ref, w1_ref, b1_ref, w2_ref, b2_ref, out_ref,
                 xbuf, sems, *, n_nodes, n_feat, bb, inv_temp, sinkhorn_iters):
    N = n_nodes
    F = n_feat
    L = N * bb
    g = pl.program_id(0)
    n_blocks = pl.num_programs(0)
    slot = jax.lax.rem(g, 2)

    def issue(blk, buf):
        for s in range(2):
            for n in range(N):
                k = s * N + n
                pltpu.make_async_copy(
                    x_hbm.at[blk, :, s, n, :], xbuf.at[buf, k],
                    sems.at[buf]).start()

    @pl.when(g == 0)
    def _():
        issue(0, 0)

    nxt = jnp.minimum(g + 1, n_blocks - 1)
    nxt_slot = jax.lax.rem(g + 1, 2)
    issue(nxt, nxt_slot)

    for k in range(2 * N):
        pltpu.make_async_copy(xbuf.at[slot, k], xbuf.at[slot, k],
                              sems.at[slot]).wait()

    parts = []
    for k in range(2 * N):
        parts.append(xbuf[slot, k].T)
    X = jnp.concatenate(parts, axis=1)

    mparts = []
    for s in range(2):
        szs = sz_ref[0, s:s + 1, :]
        for n in range(N):
            mparts.append(jnp.where(float(n) < szs, 1.0, 0.0))
    mrow = jnp.concatenate(mparts, axis=1)

    h = jnp.maximum(
        jnp.dot(w1_ref[...], X, preferred_element_type=jnp.float32) + b1_ref[...], 0.0)
    e = jnp.dot(w2_ref[...], h, preferred_element_type=jnp.float32) + b2_ref[...]
    e = e * mrow

    ce = e[:, L:]
    rows = []
    for n in range(N):
        qr = pltpu.repeat(e[:, n * bb:(n + 1) * bb], N, axis=1)
        rows.append(jnp.sum(qr * ce, axis=0, keepdims=True))
    la = jnp.concatenate(rows, axis=0) * inv_temp

    for _ in range(sinkhorn_iters):
        la = la - _lse_lane_tiles(la, N, bb)
        la = la - _lse_sublanes(la)
    plan = jnp.exp(la)

    Xb = X.astype(jnp.bfloat16)
    pb = plan.astype(jnp.bfloat16)
    sc = jnp.zeros((1, bb), jnp.float32)
    for n in range(N):
        pn = pb[n:n + 1, :]
        recon = pn[:, 0:bb] * Xb[:, L:L + bb]
        for m in range(1, N):
            recon = recon + pn[:, m * bb:(m + 1) * bb] * Xb[:, L + m * bb:L + (m + 1) * bb]
        diff = jnp.maximum(Xb[:, n * bb:(n + 1) * bb] - recon, jnp.bfloat16(0.0))
        part = jnp.sum(diff, axis=0, keepdims=True, dtype=jnp.bfloat16)
        sc = sc + part.astype(jnp.float32)
    out_ref[...] = (-sc).reshape(1, 1, bb)

    @pl.when(g == n_blocks - 1)
    def _():
        for k in range(2 * N):
            pltpu.make_async_copy(xbuf.at[nxt_slot, k], xbuf.at[nxt_slot, k],
                                  sems.at[nxt_slot]).wait()


def _pick_batch_block(batch):
    for d in (128, 64, 32, 16, 8, 4, 2):
        if d <= batch and batch % d == 0 and batch // d >= 2:
            return d
    return batch


def _run_blocks(x, sz, w1t, b1c, w2t, b2c, *, n_nodes, n_feat, bb):
    G = x.shape[0]
    T = w1t.shape[0]
    F = n_feat
    N = n_nodes
    kern = functools.partial(_nanl_kernel, n_nodes=N, n_feat=F, bb=bb,
                             inv_temp=10.0, sinkhorn_iters=10)
    return pl.pallas_call(
        kern,
        grid=(G,),
        out_shape=jax.ShapeDtypeStruct((G, 1, bb), jnp.float32),
        in_specs=[
            pl.BlockSpec(memory_space=pl.ANY),
            pl.BlockSpec((1, 2, bb), lambda g: (g, 0, 0)),
            pl.BlockSpec((T, F), lambda g: (0, 0)),
            pl.BlockSpec((T, 1), lambda g: (0, 0)),
            pl.BlockSpec((T, T), lambda g: (0, 0)),
            pl.BlockSpec((T, 1), lambda g: (0, 0)),
        ],
        out_specs=pl.BlockSpec((1, 1, bb), lambda g: (g, 0, 0)),
        scratch_shapes=[
            pltpu.VMEM((2, 2 * N, bb, F), jnp.float32),
            pltpu.SemaphoreType.DMA((2,)),
        ],
        compiler_params=pltpu.CompilerParams(
            dimension_semantics=("arbitrary",),
            vmem_limit_bytes=32 * 1024 * 1024),
    )(x, sz, w1t, b1c, w2t, b2c)


def kernel(stacked_qc, graph_sizes, w1, b1, w2, b2):
    B, two, N, F = stacked_qc.shape
    assert two == 2
    bb = _pick_batch_block(B)
    G = B // bb

    x = stacked_qc.astype(jnp.float32).reshape(G, bb, 2, N, F)
    sz = (graph_sizes.astype(jnp.float32)
          .reshape(G, bb, 2)
          .transpose(0, 2, 1))
    wargs = (w1.T.astype(jnp.float32), b1.reshape(-1, 1).astype(jnp.float32),
             w2.T.astype(jnp.float32), b2.reshape(-1, 1).astype(jnp.float32))

    run = functools.partial(_run_blocks, n_nodes=N, n_feat=F, bb=bb)
    out = run(x, sz, *wargs)
    return out.reshape(B)
